# baseline (speedup 1.0000x reference)
import functools

import jax
import jax.numpy as jnp
from jax import lax
from jax.experimental import pallas as pl
from jax.experimental.pallas import tpu as pltpu
from jax.experimental.pallas import tpu_sc as plsc

NC = 2
NS = 16
L = 16
NW = NC * NS

B = 16384
E = 64
W = 2 * E
CH = 128
BPW = B // NW
NCH = BPW // CH

MIN_RATING = 0.5
MAX_RATING = 5.0


def _body(uidx_hbm, midx_hbm, tab_hbm, ub_hbm, mb_hbm, out_hbm,
          uidx_v, midx_v, uidx2_v, midx2_v,
          urows_v, mrows_v, ub_v, mb_v, out_v, sem0, sem1, semb):
    wid = lax.axis_index("s") * NC + lax.axis_index("c")
    base = wid * BPW

    pltpu.sync_copy(uidx_hbm.at[pl.ds(base, BPW)], uidx_v)
    pltpu.sync_copy(midx_hbm.at[pl.ds(base, BPW)], midx_v)
    for j in range(NCH):
        for c in range(CH // L):
            uidx2_v[j, pl.ds(c * L, L)] = uidx_v[pl.ds(j * CH + c * L, L)]
            midx2_v[j, pl.ds(c * L, L)] = midx_v[pl.ds(j * CH + c * L, L)]

    bias_copies = []
    for j in range(NCH):
        bias_copies.append(pltpu.async_copy(
            ub_hbm.at[uidx2_v.at[j]], ub_v.at[pl.ds(j * CH, CH)], semb))
        bias_copies.append(pltpu.async_copy(
            mb_hbm.at[midx2_v.at[j]], mb_v.at[pl.ds(j * CH, CH)], semb))

    lanes = lax.iota(jnp.int32, L)
    scale = MAX_RATING - MIN_RATING
    sems = [sem0, sem1]

    def fire_chunk(j):
        s = sems[j % 2]
        cu = pltpu.async_copy(tab_hbm.at[uidx2_v.at[j]], urows_v.at[j % 2], s)
        cm = pltpu.async_copy(tab_hbm.at[midx2_v.at[j]], mrows_v.at[j % 2], s)
        return [cu, cm]

    inflight = fire_chunk(0)
    for j in range(NCH):
        nxt = fire_chunk(j + 1) if j + 1 < NCH else None
        for c in inflight:
            c.wait()
        bufsel = jnp.full((L,), j % 2, jnp.int32)
        for g in range(CH // L):
            r0 = j * CH + g * L
            tvec = lanes + g * L

            def kstep(k4, acc):
                for t in range(4):
                    kv = (lanes + (k4 * 4 + t)) & (E - 1)
                    u = plsc.load_gather(urows_v, [bufsel, tvec, kv])
                    m = plsc.load_gather(mrows_v, [bufsel, tvec, kv + E])
                    acc = acc + u * m
                return acc

            acc = lax.fori_loop(0, E // 4, kstep, jnp.zeros((L,), jnp.float32))
            out_v[pl.ds(r0, L)] = acc
        inflight = nxt

    for c in bias_copies:
        c.wait()
    for g in range(BPW // L):
        r0 = g * L
        x = out_v[pl.ds(r0, L)] + ub_v[pl.ds(r0, L)] + mb_v[pl.ds(r0, L)]
        y = 1.0 / (1.0 + jnp.exp(-x))
        out_v[pl.ds(r0, L)] = y * scale + MIN_RATING

    pltpu.sync_copy(out_v, out_hbm.at[pl.ds(base, BPW)])


_sc_call = functools.partial(
    pl.kernel,
    out_type=jax.ShapeDtypeStruct((B,), jnp.float32),
    mesh=plsc.VectorSubcoreMesh(core_axis_name="c", subcore_axis_name="s"),
    compiler_params=pltpu.CompilerParams(
        needs_layout_passes=False, use_tc_tiling_on_sc=True),
    scratch_types=[
        pltpu.VMEM((BPW,), jnp.int32),
        pltpu.VMEM((BPW,), jnp.int32),
        pltpu.VMEM((NCH, CH), jnp.int32),
        pltpu.VMEM((NCH, CH), jnp.int32),
        pltpu.VMEM((2, CH, W), jnp.float32),
        pltpu.VMEM((2, CH, W), jnp.float32),
        pltpu.VMEM((BPW,), jnp.float32),
        pltpu.VMEM((BPW,), jnp.float32),
        pltpu.VMEM((BPW,), jnp.float32),
        pltpu.SemaphoreType.DMA,
        pltpu.SemaphoreType.DMA,
        pltpu.SemaphoreType.DMA,
    ],
)(_body)


@jax.jit
def kernel(inputs, user_emb, user_bias, movie_emb, movie_bias):
    n = min(user_emb.shape[0], movie_emb.shape[0])
    h = n // 2
    u1 = lax.optimization_barrier(user_emb[:h])
    u2 = lax.optimization_barrier(user_emb[h:n])
    m1 = lax.optimization_barrier(movie_emb[:h])
    m2 = lax.optimization_barrier(movie_emb[h:n])
    f1 = lax.optimization_barrier(jnp.concatenate([u1, m1], axis=1))
    f2 = lax.optimization_barrier(jnp.concatenate([u2, m2], axis=1))
    fused = jnp.concatenate([f1, f2], axis=0)
    y = _sc_call(inputs[:, 0], inputs[:, 1], fused,
                 user_bias[:n].reshape(-1), movie_bias[:n].reshape(-1))
    return y.reshape(B, 1)


# speedup vs baseline: 1.6448x; 1.6448x over previous
import functools

import jax
import jax.numpy as jnp
from jax import lax
from jax.experimental import pallas as pl
from jax.experimental.pallas import tpu as pltpu
from jax.experimental.pallas import tpu_sc as plsc

NC = 2
NS = 16
L = 16
NW = NC * NS

B = 16384
E = 64
W = 2 * E
CH = 128
BPW = B // NW
NCH = BPW // CH

MIN_RATING = 0.5
MAX_RATING = 5.0


def _body(uidx_hbm, midx_hbm, tab_hbm, ub_hbm, mb_hbm, out_hbm,
          uidx_v, midx_v, uidx2_v, midx2_v,
          urows_v, mrows_v, ub_v, mb_v, out_v, sem0, sem1, semb):
    wid = lax.axis_index("s") * NC + lax.axis_index("c")
    base = wid * BPW

    pltpu.sync_copy(uidx_hbm.at[pl.ds(base, BPW)], uidx_v)
    pltpu.sync_copy(midx_hbm.at[pl.ds(base, BPW)], midx_v)
    for j in range(NCH):
        for c in range(CH // L):
            uidx2_v[j, pl.ds(c * L, L)] = uidx_v[pl.ds(j * CH + c * L, L)]
            midx2_v[j, pl.ds(c * L, L)] = midx_v[pl.ds(j * CH + c * L, L)]

    bias_copies = []
    for j in range(NCH):
        bias_copies.append(pltpu.async_copy(
            ub_hbm.at[uidx2_v.at[j]], ub_v.at[pl.ds(j * CH, CH)], semb))
        bias_copies.append(pltpu.async_copy(
            mb_hbm.at[midx2_v.at[j]], mb_v.at[pl.ds(j * CH, CH)], semb))

    lanes = lax.iota(jnp.int32, L)
    scale = MAX_RATING - MIN_RATING
    sems = [sem0, sem1]

    def fire_chunk(j):
        s = sems[j % 2]
        cu = pltpu.async_copy(tab_hbm.at[uidx2_v.at[j]], urows_v.at[j % 2], s)
        cm = pltpu.async_copy(tab_hbm.at[midx2_v.at[j]], mrows_v.at[j % 2], s)
        return [cu, cm]

    inflight = fire_chunk(0)
    for j in range(NCH):
        nxt = fire_chunk(j + 1) if j + 1 < NCH else None
        for c in inflight:
            c.wait()
        bufsel = jnp.full((L,), j % 2, jnp.int32)
        for g in range(CH // L):
            r0 = j * CH + g * L
            tvec = lanes + g * L

            def kstep(k4, acc):
                for t in range(4):
                    kv = (lanes + (k4 * 4 + t)) & (E - 1)
                    u = plsc.load_gather(urows_v, [bufsel, tvec, kv])
                    m = plsc.load_gather(mrows_v, [bufsel, tvec, kv + E])
                    acc = acc + u * m
                return acc

            acc = lax.fori_loop(0, E // 4, kstep, jnp.zeros((L,), jnp.float32))
            out_v[pl.ds(r0, L)] = acc
        inflight = nxt

    for c in bias_copies:
        c.wait()
    for g in range(BPW // L):
        r0 = g * L
        x = out_v[pl.ds(r0, L)] + ub_v[pl.ds(r0, L)] + mb_v[pl.ds(r0, L)]
        y = 1.0 / (1.0 + jnp.exp(-x))
        out_v[pl.ds(r0, L)] = y * scale + MIN_RATING

    pltpu.sync_copy(out_v, out_hbm.at[pl.ds(base, BPW)])


_sc_call = functools.partial(
    pl.kernel,
    out_type=jax.ShapeDtypeStruct((B,), jnp.float32),
    mesh=plsc.VectorSubcoreMesh(core_axis_name="c", subcore_axis_name="s"),
    compiler_params=pltpu.CompilerParams(
        needs_layout_passes=False, use_tc_tiling_on_sc=True),
    scratch_types=[
        pltpu.VMEM((BPW,), jnp.int32),
        pltpu.VMEM((BPW,), jnp.int32),
        pltpu.VMEM((NCH, CH), jnp.int32),
        pltpu.VMEM((NCH, CH), jnp.int32),
        pltpu.VMEM((2, CH, W), jnp.float32),
        pltpu.VMEM((2, CH, W), jnp.float32),
        pltpu.VMEM((BPW,), jnp.float32),
        pltpu.VMEM((BPW,), jnp.float32),
        pltpu.VMEM((BPW,), jnp.float32),
        pltpu.SemaphoreType.DMA,
        pltpu.SemaphoreType.DMA,
        pltpu.SemaphoreType.DMA,
    ],
)(_body)


@jax.jit
def kernel(inputs, user_emb, user_bias, movie_emb, movie_bias):
    n = min(user_emb.shape[0], movie_emb.shape[0])
    ue = lax.optimization_barrier(user_emb[:n])
    fused = jnp.concatenate([ue, movie_emb[:n]], axis=1)
    y = _sc_call(inputs[:, 0], inputs[:, 1], fused,
                 user_bias[:n].reshape(-1), movie_bias[:n].reshape(-1))
    return y.reshape(B, 1)
